# baseline (speedup 1.0000x reference)
import dataclasses
import functools

import jax
import jax.numpy as jnp
from jax import lax
from jax.experimental import pallas as pl
from jax.experimental.pallas import tpu as pltpu
from jax.experimental.pallas import tpu_sc as plsc

N = 10000
E = 160000
D = 256
NC = 2
NS = 16
LANES = 16
HALF = N // NC
QTR = HALF // 2
PASSES = 2

EPT = E // NS
SEDGES = 1024
SPHASES = 10

CH = 64
CAP = 10368

BM = 1000
PACK_SHIFT = 14


def _mm_body(x_r, w_r, b_r, o_r):
    h = jnp.dot(x_r[...], w_r[...], preferred_element_type=jnp.float32)
    o_r[...] = h + b_r[...]


def _linear(x, W, b2):
    return pl.pallas_call(
        _mm_body,
        grid=(N // BM,),
        in_specs=[
            pl.BlockSpec((BM, D), lambda i: (i, 0)),
            pl.BlockSpec((D, D), lambda i: (0, 0)),
            pl.BlockSpec((1, D), lambda i: (0, 0)),
        ],
        out_specs=pl.BlockSpec((BM, D), lambda i: (i, 0)),
        out_shape=jax.ShapeDtypeStruct((N, D), jnp.float32),
    )(x, W, b2)


def _sc_body(h_hbm, src_hbm, dst_hbm, w_hbm, out_hbm,
             gbuf0, gbuf1, sbuf0, sbuf1, src_st, dst_st, w_st,
             cpk_v, cw_v, cidx0, cidx1, cdst0, cdst1,
             acc_sh, g0, g1, s0, s1):
    c = lax.axis_index("c")
    s = lax.axis_index("s")
    base_edge = s * EPT

    zero = jnp.zeros((LANES,), jnp.float32)
    izero = jnp.zeros((LANES,), jnp.int32)
    iota2 = lax.iota(jnp.int32, LANES) * 2

    ACC_ROWS = 2 * QTR
    n_zch = ACC_ROWS // (2 * CH) + 1
    z_tail = ACC_ROWS - (n_zch - 1) * (2 * CH)

    def _for_each_owned_chunk(fn):
        for j in range((n_zch + NS - 1) // NS):
            i = s + NS * j

            @pl.when(i < n_zch - 1)
            def _():
                fn(i * 2 * CH, 2 * CH)

            @pl.when(i == n_zch - 1)
            def _():
                fn(i * 2 * CH, z_tail)

    def _drain(sem):
        pltpu.make_async_copy(h_hbm.at[pl.ds(0, CH)], gbuf0, sem).wait()

    for q in range(PASSES):
        lo = c * HALF + q * QTR
        lo_v = jnp.full((LANES,), lo, jnp.int32)
        hi_v = jnp.full((LANES,), lo + QTR, jnp.int32)

        @pl.loop(0, CAP, step=LANES)
        def _(i):
            cpk_v.at[pl.ds(i, LANES)][...] = izero
            cw_v.at[pl.ds(i, LANES)][...] = zero

        def _scan_phase(p, pos):
            st_base = base_edge + p * SEDGES
            n_e = min(SEDGES, EPT - p * SEDGES)
            pltpu.sync_copy(src_hbm.at[pl.ds(st_base, n_e)],
                            src_st.at[pl.ds(0, n_e)])
            pltpu.sync_copy(dst_hbm.at[pl.ds(st_base, n_e)],
                            dst_st.at[pl.ds(0, n_e)])
            pltpu.sync_copy(w_hbm.at[pl.ds(st_base, n_e)],
                            w_st.at[pl.ds(0, n_e)])

            def _group(g, pos):
                j = g * LANES
                srcv = src_st.at[pl.ds(j, LANES)][...]
                dstv = dst_st.at[pl.ds(j, LANES)][...]
                wv = w_st.at[pl.ds(j, LANES)][...]
                m = (dstv >= lo_v) & (dstv < hi_v)
                pk = srcv | ((dstv - lo_v) << PACK_SHIFT)
                plsc.store_compressed(cpk_v.at[pl.ds(pos, LANES)], pk,
                                      mask=m)
                plsc.store_compressed(cw_v.at[pl.ds(pos, LANES)], wv,
                                      mask=m)
                return pos + jnp.sum(m.astype(jnp.int32))

            return lax.fori_loop(0, n_e // LANES, _group, pos)

        pos = jnp.int32(0)
        for p in range(SPHASES):
            pos = _scan_phase(p, pos)

        @pl.loop(0, 2 * CH)
        def _(e):
            for r in range(128 // LANES):
                sbuf0.at[e, pl.ds(r * LANES, LANES)][...] = zero

        _for_each_owned_chunk(
            lambda r0, n: pltpu.sync_copy(sbuf0.at[pl.ds(0, n)],
                                          acc_sh.at[pl.ds(r0, n)]))

        plsc.subcore_barrier()

        nch = jnp.maximum(((pos + 2 * CH - 1) // (2 * CH)) * 2, 2)

        def _unpack(t, cidx, cdst):
            for j in range(CH // LANES):
                pk = cpk_v.at[pl.ds(t * CH + j * LANES, LANES)][...]
                cidx.at[pl.ds(j * LANES, LANES)][...] = (
                    pk & ((1 << PACK_SHIFT) - 1))
                d2 = (pk >> PACK_SHIFT) * 2
                base = iota2 + (j * 2 * LANES)
                plsc.store_scatter(cdst, [base], d2)
                plsc.store_scatter(cdst, [base + 1], d2 + 1)

        def _start_gather(cidx, gbuf, sem):
            pltpu.async_copy(h_hbm.at[cidx], gbuf, sem)

        def _start_scatter(cdst, sbuf, sem):
            pltpu.async_copy(sbuf, acc_sh.at[cdst], sem, add=True)

        def _scale(t, gbuf, sbuf):
            @plsc.parallel_loop(0, CH, unroll=2)
            def _(e):
                ef = jnp.full((LANES,), t * CH + e, jnp.int32)
                wv = plsc.load_gather(cw_v, [ef])
                for r in range(D // LANES):
                    src_slc = (e, pl.ds(r * LANES, LANES))
                    dst_slc = (2 * e + r // 8, pl.ds((r % 8) * LANES, LANES))
                    sbuf.at[dst_slc][...] = gbuf.at[src_slc][...] * wv

        _unpack(0, cidx0, cdst0)
        _start_gather(cidx0, gbuf0, g0)

        @pl.loop(0, nch, step=2)
        def _(t):
            _drain(g0)

            @pl.when(t > 0)
            def _():
                _drain(s1)

            _unpack(t + 1, cidx1, cdst1)
            _start_gather(cidx1, gbuf1, g1)
            _scale(t, gbuf0, sbuf0)
            _start_scatter(cdst0, sbuf0, s0)
            _drain(g1)
            _scale(t + 1, gbuf1, sbuf1)
            _drain(s0)

            @pl.when(t + 2 < nch)
            def _():
                _unpack(t + 2, cidx0, cdst0)
                _start_gather(cidx0, gbuf0, g0)

            _start_scatter(cdst1, sbuf1, s1)

        _drain(s1)

        plsc.subcore_barrier()

        def _relu_out(r0, n):
            pltpu.sync_copy(acc_sh.at[pl.ds(r0, n)], sbuf0.at[pl.ds(0, n)])

            @pl.loop(0, n)
            def _(e):
                for r in range(128 // LANES):
                    slc = (e, pl.ds(r * LANES, LANES))
                    sbuf0.at[slc][...] = jnp.maximum(sbuf0.at[slc][...], 0.0)

            pltpu.sync_copy(sbuf0.at[pl.ds(0, n)],
                            out_hbm.at[pl.ds(2 * lo + r0, n)])

        _for_each_owned_chunk(_relu_out)

        if q + 1 < PASSES:
            plsc.subcore_barrier()


@functools.lru_cache(maxsize=1)
def _sc_message_passing():
    cp = pltpu.CompilerParams()
    if "needs_layout_passes" in pltpu.CompilerParams.__dataclass_fields__:
        cp = dataclasses.replace(cp, needs_layout_passes=False)
    return pl.kernel(
        _sc_body,
        compiler_params=cp,
        out_type=jax.ShapeDtypeStruct((2 * N, 128), jnp.float32),
        mesh=plsc.VectorSubcoreMesh(core_axis_name="c", subcore_axis_name="s",
                                    num_cores=NC, num_subcores=NS),
        scratch_types=[
            pltpu.VMEM((CH, D), jnp.float32),
            pltpu.VMEM((CH, D), jnp.float32),
            pltpu.VMEM((2 * CH, 128), jnp.float32),
            pltpu.VMEM((2 * CH, 128), jnp.float32),
            pltpu.VMEM((SEDGES,), jnp.int32),
            pltpu.VMEM((SEDGES,), jnp.int32),
            pltpu.VMEM((SEDGES,), jnp.float32),
            pltpu.VMEM((CAP,), jnp.int32),
            pltpu.VMEM((CAP,), jnp.float32),
            pltpu.VMEM((CH,), jnp.int32),
            pltpu.VMEM((CH,), jnp.int32),
            pltpu.VMEM((2 * CH,), jnp.int32),
            pltpu.VMEM((2 * CH,), jnp.int32),
            pltpu.VMEM_SHARED((2 * QTR, 128), jnp.float32),
            pltpu.SemaphoreType.DMA,
            pltpu.SemaphoreType.DMA,
            pltpu.SemaphoreType.DMA,
            pltpu.SemaphoreType.DMA,
        ],
    )


@jax.jit
def kernel(x, edge_index, edge_weight, W, b):
    h = _linear(x, W, b.reshape(1, D))

    src = edge_index[0].astype(jnp.int32)
    dst = edge_index[1].astype(jnp.int32)
    w = edge_weight.astype(jnp.float32)

    out_raw = _sc_message_passing()(h, src, dst, w)
    return out_raw.reshape(N, D)
